# speedup vs baseline: 2.9671x; 2.9671x over previous
import jax
import jax.numpy as jnp
from jax import lax
from jax.experimental import pallas as pl
from jax.experimental.pallas import tpu as pltpu
from jax.experimental.pallas import tpu_sc as plsc

NUM_TABLE_ROWS = 100000
DIM = 128
BATCH = 4096 * 50
NUM_WORKERS = 32
PER_WORKER = BATCH // NUM_WORKERS
CHUNK = 128
N_CHUNKS = PER_WORKER // CHUNK


def _emb_kernel(ids_hbm, table_hbm, out_hbm, idx_v, rows_v, gsem, ssem):
    wid = lax.axis_index("s") * 2 + lax.axis_index("c")
    base = wid * PER_WORKER
    pltpu.sync_copy(ids_hbm.at[pl.ds(base, PER_WORKER)], idx_v)

    @pl.loop(0, N_CHUNKS)
    def _chunk(i):
        idx = idx_v.at[pl.ds(i * CHUNK, CHUNK)]
        pltpu.async_copy(table_hbm.at[idx], rows_v, gsem).wait()
        pltpu.sync_copy(rows_v, out_hbm.at[pl.ds(base + i * CHUNK, CHUNK)])


@jax.jit
def _lookup(ids_flat, embeddings):
    mesh = plsc.VectorSubcoreMesh(core_axis_name="c", subcore_axis_name="s")
    return pl.kernel(
        _emb_kernel,
        out_type=jax.ShapeDtypeStruct((BATCH, DIM), jnp.float32),
        mesh=mesh,
        scratch_types=[
            pltpu.VMEM((PER_WORKER,), jnp.int32),
            pltpu.VMEM((CHUNK, DIM), jnp.float32),
            pltpu.SemaphoreType.DMA,
            pltpu.SemaphoreType.DMA,
        ],
    )(ids_flat, embeddings)


def kernel(token_ids, embeddings):
    b, t = token_ids.shape
    out = _lookup(token_ids.reshape(-1), embeddings)
    return out.reshape(b, t, DIM)


# speedup vs baseline: 3.3293x; 1.1221x over previous
import jax
import jax.numpy as jnp
from jax import lax
from jax.experimental import pallas as pl
from jax.experimental.pallas import tpu as pltpu
from jax.experimental.pallas import tpu_sc as plsc

NUM_TABLE_ROWS = 100000
DIM = 128
BATCH = 4096 * 50
NUM_WORKERS = 32
PER_WORKER = BATCH // NUM_WORKERS
CHUNK = 128
N_CHUNKS = PER_WORKER // CHUNK
NBUF = 2


def _emb_kernel(ids_hbm, table_hbm, out_hbm, idx_v, rows0, rows1, g0, g1,
                s0, s1):
    rows = [rows0, rows1]
    gsem = [g0, g1]
    ssem = [s0, s1]
    wid = lax.axis_index("s") * 2 + lax.axis_index("c")
    base = wid * PER_WORKER
    pltpu.sync_copy(ids_hbm.at[pl.ds(base, PER_WORKER)], idx_v)

    def g_copy(c, b):
        idx = idx_v.at[pl.ds(c * CHUNK, CHUNK)]
        return pltpu.make_async_copy(table_hbm.at[idx], rows[b], gsem[b])

    def s_copy(c, b):
        dst = out_hbm.at[pl.ds(base + c * CHUNK, CHUNK)]
        return pltpu.make_async_copy(rows[b], dst, ssem[b])

    g_copy(0, 0).start()

    @pl.loop(0, N_CHUNKS, step=NBUF)
    def _step(i):
        for b in range(NBUF):
            c = i + b
            nb = (b + 1) % NBUF

            @pl.when(c + 1 < N_CHUNKS)
            def _prefetch():
                @pl.when(c >= 1)
                def _drain_prev():
                    s_copy(c - 1, nb).wait()
                g_copy(c + 1, nb).start()

            g_copy(c, b).wait()
            s_copy(c, b).start()

    s_copy(N_CHUNKS - 2, (N_CHUNKS - 2) % NBUF).wait()
    s_copy(N_CHUNKS - 1, (N_CHUNKS - 1) % NBUF).wait()


@jax.jit
def _lookup(ids_flat, embeddings):
    mesh = plsc.VectorSubcoreMesh(core_axis_name="c", subcore_axis_name="s")
    return pl.kernel(
        _emb_kernel,
        out_type=jax.ShapeDtypeStruct((BATCH, DIM), jnp.float32),
        mesh=mesh,
        scratch_types=[
            pltpu.VMEM((PER_WORKER,), jnp.int32),
            pltpu.VMEM((CHUNK, DIM), jnp.float32),
            pltpu.VMEM((CHUNK, DIM), jnp.float32),
            pltpu.SemaphoreType.DMA,
            pltpu.SemaphoreType.DMA,
            pltpu.SemaphoreType.DMA,
            pltpu.SemaphoreType.DMA,
        ],
    )(ids_flat, embeddings)


def kernel(token_ids, embeddings):
    b, t = token_ids.shape
    out = _lookup(token_ids.reshape(-1), embeddings)
    return out.reshape(b, t, DIM)


# speedup vs baseline: 3.3413x; 1.0036x over previous
import jax
import jax.numpy as jnp
from jax import lax
from jax.experimental import pallas as pl
from jax.experimental.pallas import tpu as pltpu
from jax.experimental.pallas import tpu_sc as plsc

NUM_TABLE_ROWS = 100000
DIM = 128
BATCH = 4096 * 50
NUM_WORKERS = 32
PER_WORKER = BATCH // NUM_WORKERS
CHUNK = 128
N_CHUNKS = PER_WORKER // CHUNK
NBUF = 5


def _emb_kernel(ids_hbm, table_hbm, out_hbm, idx_v, *bufs):
    rows = list(bufs[:NBUF])
    gsem = list(bufs[NBUF:2 * NBUF])
    ssem = list(bufs[2 * NBUF:])
    wid = lax.axis_index("s") * 2 + lax.axis_index("c")
    base = wid * PER_WORKER
    pltpu.sync_copy(ids_hbm.at[pl.ds(base, PER_WORKER)], idx_v)

    def g_copy(c, b):
        idx = idx_v.at[pl.ds(c * CHUNK, CHUNK)]
        return pltpu.make_async_copy(table_hbm.at[idx], rows[b], gsem[b])

    def s_copy(c, b):
        dst = out_hbm.at[pl.ds(base + c * CHUNK, CHUNK)]
        return pltpu.make_async_copy(rows[b], dst, ssem[b])

    for b in range(NBUF - 1):
        g_copy(b, b).start()

    @pl.loop(0, N_CHUNKS, step=NBUF)
    def _step(i):
        for b in range(NBUF):
            c = i + b
            tb = (b + NBUF - 1) % NBUF

            @pl.when(c + NBUF - 1 < N_CHUNKS)
            def _prefetch():
                @pl.when(c >= 1)
                def _drain_prev():
                    s_copy(c - 1, tb).wait()
                g_copy(c + NBUF - 1, tb).start()

            g_copy(c, b).wait()
            s_copy(c, b).start()

    for k in range(NBUF):
        c = N_CHUNKS - NBUF + k
        s_copy(c, c % NBUF).wait()


@jax.jit
def _lookup(ids_flat, embeddings):
    mesh = plsc.VectorSubcoreMesh(core_axis_name="c", subcore_axis_name="s")
    return pl.kernel(
        _emb_kernel,
        out_type=jax.ShapeDtypeStruct((BATCH, DIM), jnp.float32),
        mesh=mesh,
        scratch_types=(
            [pltpu.VMEM((PER_WORKER,), jnp.int32)]
            + [pltpu.VMEM((CHUNK, DIM), jnp.float32)] * NBUF
            + [pltpu.SemaphoreType.DMA] * (2 * NBUF)
        ),
    )(ids_flat, embeddings)


def kernel(token_ids, embeddings):
    b, t = token_ids.shape
    out = _lookup(token_ids.reshape(-1), embeddings)
    return out.reshape(b, t, DIM)
